# baseline (speedup 1.0000x reference)
import functools

import jax
import jax.numpy as jnp
from jax import lax
from jax.experimental import pallas as pl
from jax.experimental.pallas import tpu as pltpu
from jax.experimental.pallas import tpu_sc as plsc

_ALPHA = 1000.0
_M_BLK = 1024
_LANES = 16
_IDX_MASK = 0xFFF
_VAL_MASK = ~0xFFF


def _keys_body(d_ref, row_e_ref, row_idx_ref, col_e_ref, col_idx_ref):
    i = pl.program_id(1)
    nb = pl.num_programs(1)
    x = d_ref[0]
    mblk, n = x.shape

    vb = (lax.bitcast_convert_type(x, jnp.int32) + 0x800000) & _VAL_MASK
    col_iota = lax.broadcasted_iota(jnp.int32, (mblk, n), 1)
    row_iota = lax.broadcasted_iota(jnp.int32, (mblk, n), 0) + i * mblk

    rkey = lax.bitcast_convert_type(vb | col_iota, jnp.float32)
    ckey = lax.bitcast_convert_type(vb | row_iota, jnp.float32)

    rk = jnp.min(rkey, axis=1)
    rkb = lax.bitcast_convert_type(rk, jnp.int32)
    row_idx_ref[0, 0] = rkb & _IDX_MASK
    row_e_ref[0, 0] = jnp.exp(
        lax.bitcast_convert_type((rkb & _VAL_MASK) - 0x7FF800, jnp.float32)
        * (-_ALPHA)
    )

    ck = jnp.min(ckey, axis=0)

    @pl.when(i == 0)
    def _():
        col_e_ref[0, 0] = ck

    @pl.when((i > 0) & (i < nb - 1))
    def _():
        col_e_ref[0, 0] = jnp.minimum(col_e_ref[0, 0], ck)

    @pl.when(i == nb - 1)
    def _():
        ckm = jnp.minimum(col_e_ref[0, 0], ck)
        ckb = lax.bitcast_convert_type(ckm, jnp.int32)
        col_idx_ref[0, 0] = ckb & _IDX_MASK
        col_e_ref[0, 0] = jnp.exp(
            lax.bitcast_convert_type((ckb & _VAL_MASK) - 0x7FF800, jnp.float32)
            * (-_ALPHA)
        )


def _stage1(D, interpret=False):
    B, M, N = D.shape
    nb = M // _M_BLK
    return pl.pallas_call(
        _keys_body,
        grid=(B, nb),
        in_specs=[pl.BlockSpec((1, _M_BLK, N), lambda b, i: (b, i, 0))],
        out_specs=[
            pl.BlockSpec((1, 1, _M_BLK), lambda b, i: (b, 0, i)),
            pl.BlockSpec((1, 1, _M_BLK), lambda b, i: (b, 0, i)),
            pl.BlockSpec((1, 1, N), lambda b, i: (b, 0, 0)),
            pl.BlockSpec((1, 1, N), lambda b, i: (b, 0, 0)),
        ],
        out_shape=[
            jax.ShapeDtypeStruct((B, 1, M), jnp.float32),
            jax.ShapeDtypeStruct((B, 1, M), jnp.int32),
            jax.ShapeDtypeStruct((B, 1, N), jnp.float32),
            jax.ShapeDtypeStruct((B, 1, N), jnp.int32),
        ],
        compiler_params=pltpu.CompilerParams(
            dimension_semantics=("parallel", "arbitrary"),
        ),
        interpret=interpret,
    )(D)


_UNROLL = 4


def _stage2_sc(idx_all, val_all, nbins):
    P, M = idx_all.shape
    mesh = plsc.VectorSubcoreMesh(core_axis_name="c", subcore_axis_name="s")

    @functools.partial(
        pl.kernel,
        mesh=mesh,
        out_type=jax.ShapeDtypeStruct((P, _LANES), jnp.float32),
        scratch_types=[
            pltpu.VMEM((M,), jnp.int32),
            pltpu.VMEM((M,), jnp.float32),
            pltpu.VMEM((nbins,), jnp.float32),
            pltpu.VMEM((nbins,), jnp.float32),
            pltpu.VMEM((_LANES,), jnp.float32),
        ],
        compiler_params=pltpu.CompilerParams(needs_layout_passes=False),
    )
    def k(idx_hbm, val_hbm, out_hbm, idx_v, val_v, s_v, c_v, acc_v):
        w = lax.axis_index("s") * 2 + lax.axis_index("c")

        @pl.when(w < P)
        def _():
            pltpu.sync_copy(idx_hbm.at[w], idx_v)
            pltpu.sync_copy(val_hbm.at[w], val_v)

            zeros = jnp.zeros((_LANES,), jnp.float32)
            step = _LANES * _UNROLL

            def zero_body(t, carry):
                for u in range(_UNROLL):
                    s_v[pl.ds(t * step + u * _LANES, _LANES)] = zeros
                    c_v[pl.ds(t * step + u * _LANES, _LANES)] = zeros
                return carry

            lax.fori_loop(0, nbins // step, zero_body, 0)

            ones = jnp.ones((_LANES,), jnp.float32)

            def scat_body(t, carry):
                for u in range(_UNROLL):
                    o = t * step + u * _LANES
                    iv = idx_v[pl.ds(o, _LANES)]
                    vv = val_v[pl.ds(o, _LANES)]
                    plsc.addupdate_scatter(s_v, [iv], vv)
                    plsc.addupdate_scatter(c_v, [iv], ones)
                return carry

            lax.fori_loop(0, M // step, scat_body, 0)

            def red_body(t, acc):
                for u in range(_UNROLL):
                    o = t * step + u * _LANES
                    sv = s_v[pl.ds(o, _LANES)]
                    cv = c_v[pl.ds(o, _LANES)]
                    acc = acc + jnp.where(
                        cv > 0.0, sv / jnp.maximum(cv, 1.0), 0.0
                    )
                return acc

            acc = lax.fori_loop(0, nbins // step, red_body, zeros)
            acc_v[...] = acc
            pltpu.sync_copy(acc_v, out_hbm.at[w])

    return k(idx_all, val_all)


def kernel(D):
    B, M, N = D.shape
    halves = []
    hb = B // 2
    for h in range(2):
        Dh = lax.slice_in_dim(D, h * hb, (h + 1) * hb, axis=0)
        row_e, row_idx, col_e, col_idx = _stage1(Dh)
        idx_all = jnp.concatenate(
            [row_idx.reshape(hb, M), col_idx.reshape(hb, N)], axis=0
        )
        val_all = jnp.concatenate(
            [row_e.reshape(hb, M), col_e.reshape(hb, N)], axis=0
        )
        halves.append(_stage2_sc(idx_all, val_all, max(M, N)))
    return 1.0 - (jnp.sum(halves[0]) + jnp.sum(halves[1])) / (2.0 * B * M)


# speedup vs baseline: 2.2690x; 2.2690x over previous
import functools

import jax
import jax.numpy as jnp
from jax import lax
from jax.experimental import pallas as pl
from jax.experimental.pallas import tpu as pltpu
from jax.experimental.pallas import tpu_sc as plsc

_ALPHA = 1000.0
_M_BLK = 1024
_LANES = 16
_IDX_MASK = 0xFFF
_VAL_MASK = ~0xFFF


def _keys_body(d_ref, row_e_ref, row_idx_ref, col_e_ref, col_idx_ref):
    i = pl.program_id(1)
    nb = pl.num_programs(1)
    x = d_ref[0]
    mblk, n = x.shape

    vb = (lax.bitcast_convert_type(x, jnp.int32) + 0x800000) & _VAL_MASK
    col_iota = lax.broadcasted_iota(jnp.int32, (mblk, n), 1)
    row_iota = lax.broadcasted_iota(jnp.int32, (mblk, n), 0) + i * mblk

    rkey = lax.bitcast_convert_type(vb | col_iota, jnp.float32)
    ckey = lax.bitcast_convert_type(vb | row_iota, jnp.float32)

    rk = jnp.min(rkey, axis=1)
    rkb = lax.bitcast_convert_type(rk, jnp.int32)
    row_idx_ref[0, 0] = rkb & _IDX_MASK
    row_e_ref[0, 0] = jnp.exp(
        lax.bitcast_convert_type((rkb & _VAL_MASK) - 0x7FF800, jnp.float32)
        * (-_ALPHA)
    )

    ck = jnp.min(ckey, axis=0)

    @pl.when(i == 0)
    def _():
        col_e_ref[0, 0] = ck

    @pl.when((i > 0) & (i < nb - 1))
    def _():
        col_e_ref[0, 0] = jnp.minimum(col_e_ref[0, 0], ck)

    @pl.when(i == nb - 1)
    def _():
        ckm = jnp.minimum(col_e_ref[0, 0], ck)
        ckb = lax.bitcast_convert_type(ckm, jnp.int32)
        col_idx_ref[0, 0] = ckb & _IDX_MASK
        col_e_ref[0, 0] = jnp.exp(
            lax.bitcast_convert_type((ckb & _VAL_MASK) - 0x7FF800, jnp.float32)
            * (-_ALPHA)
        )


def _stage1(D, b0=0, nbatch=None, interpret=False):
    B, M, N = D.shape
    if nbatch is None:
        nbatch = B
    nb = M // _M_BLK
    return pl.pallas_call(
        _keys_body,
        grid=(nbatch, nb),
        in_specs=[pl.BlockSpec((1, _M_BLK, N), lambda b, i: (b + b0, i, 0))],
        out_specs=[
            pl.BlockSpec((1, 1, _M_BLK), lambda b, i: (b, 0, i)),
            pl.BlockSpec((1, 1, _M_BLK), lambda b, i: (b, 0, i)),
            pl.BlockSpec((1, 1, N), lambda b, i: (b, 0, 0)),
            pl.BlockSpec((1, 1, N), lambda b, i: (b, 0, 0)),
        ],
        out_shape=[
            jax.ShapeDtypeStruct((nbatch, 1, M), jnp.float32),
            jax.ShapeDtypeStruct((nbatch, 1, M), jnp.int32),
            jax.ShapeDtypeStruct((nbatch, 1, N), jnp.float32),
            jax.ShapeDtypeStruct((nbatch, 1, N), jnp.int32),
        ],
        compiler_params=pltpu.CompilerParams(
            dimension_semantics=("parallel", "arbitrary"),
        ),
        interpret=interpret,
    )(D)


_UNROLL = 4


def _stage2_sc(idx_all, val_all, nbins):
    P, M = idx_all.shape
    mesh = plsc.VectorSubcoreMesh(core_axis_name="c", subcore_axis_name="s")

    @functools.partial(
        pl.kernel,
        mesh=mesh,
        out_type=jax.ShapeDtypeStruct((P, _LANES), jnp.float32),
        scratch_types=[
            pltpu.VMEM((M,), jnp.int32),
            pltpu.VMEM((M,), jnp.float32),
            pltpu.VMEM((nbins,), jnp.float32),
            pltpu.VMEM((nbins,), jnp.float32),
            pltpu.VMEM((_LANES,), jnp.float32),
        ],
        compiler_params=pltpu.CompilerParams(needs_layout_passes=False),
    )
    def k(idx_hbm, val_hbm, out_hbm, idx_v, val_v, s_v, c_v, acc_v):
        w = lax.axis_index("s") * 2 + lax.axis_index("c")

        @pl.when(w < P)
        def _():
            pltpu.sync_copy(idx_hbm.at[w], idx_v)
            pltpu.sync_copy(val_hbm.at[w], val_v)

            zeros = jnp.zeros((_LANES,), jnp.float32)
            step = _LANES * _UNROLL

            def zero_body(t, carry):
                for u in range(_UNROLL):
                    s_v[pl.ds(t * step + u * _LANES, _LANES)] = zeros
                    c_v[pl.ds(t * step + u * _LANES, _LANES)] = zeros
                return carry

            lax.fori_loop(0, nbins // step, zero_body, 0)

            ones = jnp.ones((_LANES,), jnp.float32)

            def scat_body(t, carry):
                for u in range(_UNROLL):
                    o = t * step + u * _LANES
                    iv = idx_v[pl.ds(o, _LANES)]
                    vv = val_v[pl.ds(o, _LANES)]
                    plsc.addupdate_scatter(s_v, [iv], vv)
                    plsc.addupdate_scatter(c_v, [iv], ones)
                return carry

            lax.fori_loop(0, M // step, scat_body, 0)

            def red_body(t, acc):
                for u in range(_UNROLL):
                    o = t * step + u * _LANES
                    sv = s_v[pl.ds(o, _LANES)]
                    cv = c_v[pl.ds(o, _LANES)]
                    acc = acc + jnp.where(
                        cv > 0.0, sv / jnp.maximum(cv, 1.0), 0.0
                    )
                return acc

            acc = lax.fori_loop(0, nbins // step, red_body, zeros)
            acc_v[...] = acc
            pltpu.sync_copy(acc_v, out_hbm.at[w])

    return k(idx_all, val_all)


def kernel(D):
    B, M, N = D.shape
    halves = []
    hb = B // 2
    for h in range(2):
        row_e, row_idx, col_e, col_idx = _stage1(D, b0=h * hb, nbatch=hb)
        idx_all = jnp.concatenate(
            [row_idx.reshape(hb, M), col_idx.reshape(hb, N)], axis=0
        )
        val_all = jnp.concatenate(
            [row_e.reshape(hb, M), col_e.reshape(hb, N)], axis=0
        )
        halves.append(_stage2_sc(idx_all, val_all, max(M, N)))
    return 1.0 - (jnp.sum(halves[0]) + jnp.sum(halves[1])) / (2.0 * B * M)


# speedup vs baseline: 2.4735x; 1.0901x over previous
import functools

import jax
import jax.numpy as jnp
from jax import lax
from jax.experimental import pallas as pl
from jax.experimental.pallas import tpu as pltpu
from jax.experimental.pallas import tpu_sc as plsc

_ALPHA = 1000.0
_M_BLK = 1024
_LANES = 16
_IDX_MASK = 0xFFF
_VAL_MASK = ~0xFFF


def _keys_body(d_ref, row_e_ref, row_idx_ref, col_e_ref, col_idx_ref):
    i = pl.program_id(1)
    nb = pl.num_programs(1)
    x = d_ref[0]
    mblk, n = x.shape

    vb = (lax.bitcast_convert_type(x, jnp.int32) + 0x800000) & _VAL_MASK
    col_iota = lax.broadcasted_iota(jnp.int32, (mblk, n), 1)
    row_iota = lax.broadcasted_iota(jnp.int32, (mblk, n), 0) + i * mblk

    rkey = lax.bitcast_convert_type(vb | col_iota, jnp.float32)
    ckey = lax.bitcast_convert_type(vb | row_iota, jnp.float32)

    rk = jnp.min(rkey, axis=1)
    rkb = lax.bitcast_convert_type(rk, jnp.int32)
    row_idx_ref[0, 0] = rkb & _IDX_MASK
    row_e_ref[0, 0] = jnp.exp(
        lax.bitcast_convert_type((rkb & _VAL_MASK) - 0x7FF800, jnp.float32)
        * (-_ALPHA)
    )

    ck = jnp.min(ckey, axis=0)

    @pl.when(i == 0)
    def _():
        col_e_ref[0, 0] = ck

    @pl.when((i > 0) & (i < nb - 1))
    def _():
        col_e_ref[0, 0] = jnp.minimum(col_e_ref[0, 0], ck)

    @pl.when(i == nb - 1)
    def _():
        ckm = jnp.minimum(col_e_ref[0, 0], ck)
        ckb = lax.bitcast_convert_type(ckm, jnp.int32)
        col_idx_ref[0, 0] = ckb & _IDX_MASK
        col_e_ref[0, 0] = jnp.exp(
            lax.bitcast_convert_type((ckb & _VAL_MASK) - 0x7FF800, jnp.float32)
            * (-_ALPHA)
        )


def _stage1(D, b0=0, nbatch=None, interpret=False):
    B, M, N = D.shape
    if nbatch is None:
        nbatch = B
    nb = M // _M_BLK
    return pl.pallas_call(
        _keys_body,
        grid=(nbatch, nb),
        in_specs=[pl.BlockSpec((1, _M_BLK, N), lambda b, i: (b + b0, i, 0))],
        out_specs=[
            pl.BlockSpec((1, 1, _M_BLK), lambda b, i: (b, 0, i)),
            pl.BlockSpec((1, 1, _M_BLK), lambda b, i: (b, 0, i)),
            pl.BlockSpec((1, 1, N), lambda b, i: (b, 0, 0)),
            pl.BlockSpec((1, 1, N), lambda b, i: (b, 0, 0)),
        ],
        out_shape=[
            jax.ShapeDtypeStruct((nbatch, 1, M), jnp.float32),
            jax.ShapeDtypeStruct((nbatch, 1, M), jnp.int32),
            jax.ShapeDtypeStruct((nbatch, 1, N), jnp.float32),
            jax.ShapeDtypeStruct((nbatch, 1, N), jnp.int32),
        ],
        compiler_params=pltpu.CompilerParams(
            dimension_semantics=("parallel", "arbitrary"),
        ),
        interpret=interpret,
    )(D)


_UNROLL = 4


def _stage2_sc(idx_all, val_all, nbins):
    P, M = idx_all.shape
    mesh = plsc.VectorSubcoreMesh(core_axis_name="c", subcore_axis_name="s")

    @functools.partial(
        pl.kernel,
        mesh=mesh,
        out_type=jax.ShapeDtypeStruct((P, _LANES), jnp.float32),
        scratch_types=[
            pltpu.VMEM((M,), jnp.int32),
            pltpu.VMEM((M,), jnp.float32),
            pltpu.VMEM((nbins,), jnp.float32),
            pltpu.VMEM((nbins,), jnp.float32),
            pltpu.VMEM((_LANES,), jnp.float32),
        ],
        compiler_params=pltpu.CompilerParams(needs_layout_passes=False),
    )
    def k(idx_hbm, val_hbm, out_hbm, idx_v, val_v, s_v, c_v, acc_v):
        w = lax.axis_index("s") * 2 + lax.axis_index("c")

        @pl.when(w < P)
        def _():
            pltpu.sync_copy(idx_hbm.at[w], idx_v)
            pltpu.sync_copy(val_hbm.at[w], val_v)

            zeros = jnp.zeros((_LANES,), jnp.float32)
            ones = jnp.ones((_LANES,), jnp.float32)

            @plsc.parallel_loop(0, nbins, _LANES, unroll=_UNROLL)
            def _zero(o):
                s_v[pl.ds(o, _LANES)] = zeros
                c_v[pl.ds(o, _LANES)] = zeros

            @plsc.parallel_loop(0, M, _LANES, unroll=_UNROLL)
            def _scat(o):
                iv = idx_v[pl.ds(o, _LANES)]
                vv = val_v[pl.ds(o, _LANES)]
                plsc.addupdate_scatter(s_v, [iv], vv)
                plsc.addupdate_scatter(c_v, [iv], ones)

            @plsc.parallel_loop(0, nbins, _LANES, unroll=_UNROLL, carry=zeros)
            def acc(o, a):
                sv = s_v[pl.ds(o, _LANES)]
                cv = c_v[pl.ds(o, _LANES)]
                return a + jnp.where(cv > 0.0, sv / jnp.maximum(cv, 1.0), 0.0)

            acc_v[...] = acc
            pltpu.sync_copy(acc_v, out_hbm.at[w])

    return k(idx_all, val_all)


def kernel(D):
    B, M, N = D.shape
    row_e, row_idx, col_e, col_idx = _stage1(D)
    idx_all = jnp.concatenate(
        [row_idx.reshape(B, M), col_idx.reshape(B, N)], axis=0
    )
    val_all = jnp.concatenate(
        [row_e.reshape(B, M), col_e.reshape(B, N)], axis=0
    )
    partials = _stage2_sc(idx_all, val_all, max(M, N))
    return 1.0 - jnp.sum(partials) / (2.0 * B * M)


# speedup vs baseline: 2.5406x; 1.0271x over previous
import functools

import jax
import jax.numpy as jnp
from jax import lax
from jax.experimental import pallas as pl
from jax.experimental.pallas import tpu as pltpu
from jax.experimental.pallas import tpu_sc as plsc

_ALPHA = 1000.0
_M_BLK = 1024
_LANES = 16
_IDX_MASK = 0xFFF
_VAL_MASK = ~0xFFF


def _keys_body(d_ref, row_e_ref, row_idx_ref, col_e_ref, col_idx_ref):
    i = pl.program_id(1)
    nb = pl.num_programs(1)
    x = d_ref[0]
    mblk, n = x.shape

    vb = (lax.bitcast_convert_type(x, jnp.int32) + 0x800000) & _VAL_MASK
    col_iota = lax.broadcasted_iota(jnp.int32, (mblk, n), 1)
    row_iota = lax.broadcasted_iota(jnp.int32, (mblk, n), 0) + i * mblk

    rkey = lax.bitcast_convert_type(vb | col_iota, jnp.float32)
    ckey = lax.bitcast_convert_type(vb | row_iota, jnp.float32)

    rk = jnp.min(rkey, axis=1)
    rkb = lax.bitcast_convert_type(rk, jnp.int32)
    row_idx_ref[0, 0] = rkb & _IDX_MASK
    row_e_ref[0, 0] = jnp.exp(
        lax.bitcast_convert_type((rkb & _VAL_MASK) - 0x7FF800, jnp.float32)
        * (-_ALPHA)
    )

    ck = jnp.min(ckey, axis=0)

    @pl.when(i == 0)
    def _():
        col_e_ref[0, 0] = ck

    @pl.when((i > 0) & (i < nb - 1))
    def _():
        col_e_ref[0, 0] = jnp.minimum(col_e_ref[0, 0], ck)

    @pl.when(i == nb - 1)
    def _():
        ckm = jnp.minimum(col_e_ref[0, 0], ck)
        ckb = lax.bitcast_convert_type(ckm, jnp.int32)
        col_idx_ref[0, 0] = ckb & _IDX_MASK
        col_e_ref[0, 0] = jnp.exp(
            lax.bitcast_convert_type((ckb & _VAL_MASK) - 0x7FF800, jnp.float32)
            * (-_ALPHA)
        )


def _stage1(D, b0=0, nbatch=None, interpret=False):
    B, M, N = D.shape
    if nbatch is None:
        nbatch = B
    nb = M // _M_BLK
    return pl.pallas_call(
        _keys_body,
        grid=(nbatch, nb),
        in_specs=[pl.BlockSpec((1, _M_BLK, N), lambda b, i: (b + b0, i, 0))],
        out_specs=[
            pl.BlockSpec((1, 1, _M_BLK), lambda b, i: (b, 0, i)),
            pl.BlockSpec((1, 1, _M_BLK), lambda b, i: (b, 0, i)),
            pl.BlockSpec((1, 1, N), lambda b, i: (b, 0, 0)),
            pl.BlockSpec((1, 1, N), lambda b, i: (b, 0, 0)),
        ],
        out_shape=[
            jax.ShapeDtypeStruct((nbatch, 1, M), jnp.float32),
            jax.ShapeDtypeStruct((nbatch, 1, M), jnp.int32),
            jax.ShapeDtypeStruct((nbatch, 1, N), jnp.float32),
            jax.ShapeDtypeStruct((nbatch, 1, N), jnp.int32),
        ],
        compiler_params=pltpu.CompilerParams(
            dimension_semantics=("parallel", "arbitrary"),
        ),
        interpret=interpret,
    )(D)


_UNROLL = 4


def _stage2_sc(row_idx, row_e, col_idx, col_e, nbins):
    B, _, M = row_idx.shape
    P = 2 * B
    mesh = plsc.VectorSubcoreMesh(core_axis_name="c", subcore_axis_name="s")

    @functools.partial(
        pl.kernel,
        mesh=mesh,
        out_type=jax.ShapeDtypeStruct((P, _LANES), jnp.float32),
        scratch_types=[
            pltpu.VMEM((M,), jnp.int32),
            pltpu.VMEM((M,), jnp.float32),
            pltpu.VMEM((nbins,), jnp.float32),
            pltpu.VMEM((nbins,), jnp.float32),
            pltpu.VMEM((_LANES,), jnp.float32),
        ],
        compiler_params=pltpu.CompilerParams(needs_layout_passes=False),
    )
    def k(ri_hbm, re_hbm, ci_hbm, ce_hbm, out_hbm, idx_v, val_v, s_v, c_v,
          acc_v):
        w = lax.axis_index("s") * 2 + lax.axis_index("c")

        @pl.when(w < B)
        def _():
            pltpu.sync_copy(ri_hbm.at[w, 0], idx_v)
            pltpu.sync_copy(re_hbm.at[w, 0], val_v)

        @pl.when((w >= B) & (w < P))
        def _():
            pltpu.sync_copy(ci_hbm.at[w - B, 0], idx_v)
            pltpu.sync_copy(ce_hbm.at[w - B, 0], val_v)

        @pl.when(w < P)
        def _():

            zeros = jnp.zeros((_LANES,), jnp.float32)
            ones = jnp.ones((_LANES,), jnp.float32)

            @plsc.parallel_loop(0, nbins, _LANES, unroll=_UNROLL)
            def _zero(o):
                s_v[pl.ds(o, _LANES)] = zeros
                c_v[pl.ds(o, _LANES)] = zeros

            @plsc.parallel_loop(0, M, _LANES, unroll=_UNROLL)
            def _scat(o):
                iv = idx_v[pl.ds(o, _LANES)]
                vv = val_v[pl.ds(o, _LANES)]
                plsc.addupdate_scatter(s_v, [iv], vv)
                plsc.addupdate_scatter(c_v, [iv], ones)

            @plsc.parallel_loop(0, nbins, _LANES, unroll=_UNROLL, carry=zeros)
            def acc(o, a):
                sv = s_v[pl.ds(o, _LANES)]
                cv = c_v[pl.ds(o, _LANES)]
                return a + jnp.where(cv > 0.0, sv / jnp.maximum(cv, 1.0), 0.0)

            acc_v[...] = acc
            pltpu.sync_copy(acc_v, out_hbm.at[w])

    return k(row_idx, row_e, col_idx, col_e)


def kernel(D):
    B, M, N = D.shape
    row_e, row_idx, col_e, col_idx = _stage1(D)
    partials = _stage2_sc(row_idx, row_e, col_idx, col_e, max(M, N))
    return 1.0 - jnp.sum(partials) / (2.0 * B * M)


# speedup vs baseline: 2.5435x; 1.0011x over previous
import functools

import jax
import jax.numpy as jnp
from jax import lax
from jax.experimental import pallas as pl
from jax.experimental.pallas import tpu as pltpu
from jax.experimental.pallas import tpu_sc as plsc

_ALPHA = 1000.0
_M_BLK = 1024
_LANES = 16
_IDX_MASK = 0xFFF
_VAL_MASK = ~0xFFF


def _keys_body(d_ref, row_e_ref, row_idx_ref, col_e_ref, col_idx_ref):
    i = pl.program_id(1)
    nb = pl.num_programs(1)
    x = d_ref[0]
    mblk, n = x.shape

    vb = (lax.bitcast_convert_type(x, jnp.int32) + 0x800000) & _VAL_MASK
    col_iota = lax.broadcasted_iota(jnp.int32, (mblk, n), 1)
    row_iota = lax.broadcasted_iota(jnp.int32, (mblk, n), 0) + i * mblk

    rkey = lax.bitcast_convert_type(vb | col_iota, jnp.float32)
    ckey = lax.bitcast_convert_type(vb | row_iota, jnp.float32)

    rk = jnp.min(rkey, axis=1)
    rkb = lax.bitcast_convert_type(rk, jnp.int32)
    row_idx_ref[0, 0] = rkb & _IDX_MASK
    row_e_ref[0, 0] = jnp.exp(
        lax.bitcast_convert_type((rkb & _VAL_MASK) - 0x7FF800, jnp.float32)
        * (-_ALPHA)
    )

    ck = jnp.min(ckey, axis=0)

    @pl.when(i == 0)
    def _():
        col_e_ref[0, 0] = ck

    @pl.when((i > 0) & (i < nb - 1))
    def _():
        col_e_ref[0, 0] = jnp.minimum(col_e_ref[0, 0], ck)

    @pl.when(i == nb - 1)
    def _():
        ckm = jnp.minimum(col_e_ref[0, 0], ck)
        ckb = lax.bitcast_convert_type(ckm, jnp.int32)
        col_idx_ref[0, 0] = ckb & _IDX_MASK
        col_e_ref[0, 0] = jnp.exp(
            lax.bitcast_convert_type((ckb & _VAL_MASK) - 0x7FF800, jnp.float32)
            * (-_ALPHA)
        )


def _stage1(D, b0=0, nbatch=None, interpret=False):
    B, M, N = D.shape
    if nbatch is None:
        nbatch = B
    nb = M // _M_BLK
    return pl.pallas_call(
        _keys_body,
        grid=(nbatch, nb),
        in_specs=[pl.BlockSpec((1, _M_BLK, N), lambda b, i: (b + b0, i, 0))],
        out_specs=[
            pl.BlockSpec((1, 1, _M_BLK), lambda b, i: (b, 0, i)),
            pl.BlockSpec((1, 1, _M_BLK), lambda b, i: (b, 0, i)),
            pl.BlockSpec((1, 1, N), lambda b, i: (b, 0, 0)),
            pl.BlockSpec((1, 1, N), lambda b, i: (b, 0, 0)),
        ],
        out_shape=[
            jax.ShapeDtypeStruct((nbatch, 1, M), jnp.float32),
            jax.ShapeDtypeStruct((nbatch, 1, M), jnp.int32),
            jax.ShapeDtypeStruct((nbatch, 1, N), jnp.float32),
            jax.ShapeDtypeStruct((nbatch, 1, N), jnp.int32),
        ],
        compiler_params=pltpu.CompilerParams(
            dimension_semantics=("parallel", "arbitrary"),
        ),
        interpret=interpret,
    )(D)


_UNROLL = 8


def _stage2_sc(row_idx, row_e, col_idx, col_e, nbins):
    B, _, M = row_idx.shape
    P = 2 * B
    mesh = plsc.VectorSubcoreMesh(core_axis_name="c", subcore_axis_name="s")

    @functools.partial(
        pl.kernel,
        mesh=mesh,
        out_type=jax.ShapeDtypeStruct((P, _LANES), jnp.float32),
        scratch_types=[
            pltpu.VMEM((M,), jnp.int32),
            pltpu.VMEM((M,), jnp.float32),
            pltpu.VMEM((nbins,), jnp.float32),
            pltpu.VMEM((nbins,), jnp.float32),
            pltpu.VMEM((_LANES,), jnp.float32),
        ],
        compiler_params=pltpu.CompilerParams(needs_layout_passes=False),
    )
    def k(ri_hbm, re_hbm, ci_hbm, ce_hbm, out_hbm, idx_v, val_v, s_v, c_v,
          acc_v):
        w = lax.axis_index("s") * 2 + lax.axis_index("c")

        @pl.when(w < B)
        def _():
            pltpu.sync_copy(ri_hbm.at[w, 0], idx_v)
            pltpu.sync_copy(re_hbm.at[w, 0], val_v)

        @pl.when((w >= B) & (w < P))
        def _():
            pltpu.sync_copy(ci_hbm.at[w - B, 0], idx_v)
            pltpu.sync_copy(ce_hbm.at[w - B, 0], val_v)

        @pl.when(w < P)
        def _():

            zeros = jnp.zeros((_LANES,), jnp.float32)
            ones = jnp.ones((_LANES,), jnp.float32)

            @plsc.parallel_loop(0, nbins, _LANES, unroll=_UNROLL)
            def _zero(o):
                s_v[pl.ds(o, _LANES)] = zeros
                c_v[pl.ds(o, _LANES)] = zeros

            @plsc.parallel_loop(0, M, _LANES, unroll=_UNROLL)
            def _scat(o):
                iv = idx_v[pl.ds(o, _LANES)]
                vv = val_v[pl.ds(o, _LANES)]
                plsc.addupdate_scatter(s_v, [iv], vv)
                plsc.addupdate_scatter(c_v, [iv], ones)

            @plsc.parallel_loop(0, nbins, _LANES, unroll=_UNROLL, carry=zeros)
            def acc(o, a):
                sv = s_v[pl.ds(o, _LANES)]
                cv = c_v[pl.ds(o, _LANES)]
                return a + jnp.where(cv > 0.0, sv / jnp.maximum(cv, 1.0), 0.0)

            acc_v[...] = acc
            pltpu.sync_copy(acc_v, out_hbm.at[w])

    return k(row_idx, row_e, col_idx, col_e)


def kernel(D):
    B, M, N = D.shape
    row_e, row_idx, col_e, col_idx = _stage1(D)
    partials = _stage2_sc(row_idx, row_e, col_idx, col_e, max(M, N))
    return 1.0 - jnp.sum(partials) / (2.0 * B * M)


# speedup vs baseline: 2.5999x; 1.0222x over previous
import functools

import jax
import jax.numpy as jnp
from jax import lax
from jax.experimental import pallas as pl
from jax.experimental.pallas import tpu as pltpu
from jax.experimental.pallas import tpu_sc as plsc

_ALPHA = 1000.0
_M_BLK = 1024
_LANES = 16
_IDX_MASK = 0xFFF
_VAL_MASK = ~0xFFF


def _keys_body(d_ref, row_e_ref, row_idx_ref, col_e_ref, col_idx_ref):
    i = pl.program_id(1)
    nb = pl.num_programs(1)
    x = d_ref[0]
    mblk, n = x.shape

    vb = (lax.bitcast_convert_type(x, jnp.int32) + 0x800000) & _VAL_MASK
    col_iota = lax.broadcasted_iota(jnp.int32, (mblk, n), 1)
    row_iota = lax.broadcasted_iota(jnp.int32, (mblk, n), 0) + i * mblk

    rkey = lax.bitcast_convert_type(vb | col_iota, jnp.float32)
    ckey = lax.bitcast_convert_type(vb | row_iota, jnp.float32)

    rk = jnp.min(rkey, axis=1)
    rkb = lax.bitcast_convert_type(rk, jnp.int32)
    row_idx_ref[0, 0] = rkb & _IDX_MASK
    row_e_ref[0, 0] = jnp.exp(
        lax.bitcast_convert_type((rkb & _VAL_MASK) - 0x7FF800, jnp.float32)
        * (-_ALPHA)
    )

    ck = jnp.min(ckey, axis=0)

    @pl.when(i == 0)
    def _():
        col_e_ref[0, 0] = ck

    @pl.when((i > 0) & (i < nb - 1))
    def _():
        col_e_ref[0, 0] = jnp.minimum(col_e_ref[0, 0], ck)

    @pl.when(i == nb - 1)
    def _():
        ckm = jnp.minimum(col_e_ref[0, 0], ck)
        ckb = lax.bitcast_convert_type(ckm, jnp.int32)
        col_idx_ref[0, 0] = ckb & _IDX_MASK
        col_e_ref[0, 0] = jnp.exp(
            lax.bitcast_convert_type((ckb & _VAL_MASK) - 0x7FF800, jnp.float32)
            * (-_ALPHA)
        )


def _stage1(D, b0=0, nbatch=None, interpret=False):
    B, M, N = D.shape
    if nbatch is None:
        nbatch = B
    nb = M // _M_BLK
    return pl.pallas_call(
        _keys_body,
        grid=(nbatch, nb),
        in_specs=[pl.BlockSpec((1, _M_BLK, N), lambda b, i: (b + b0, i, 0))],
        out_specs=[
            pl.BlockSpec((1, 1, _M_BLK), lambda b, i: (b, 0, i)),
            pl.BlockSpec((1, 1, _M_BLK), lambda b, i: (b, 0, i)),
            pl.BlockSpec((1, 1, N), lambda b, i: (b, 0, 0)),
            pl.BlockSpec((1, 1, N), lambda b, i: (b, 0, 0)),
        ],
        out_shape=[
            jax.ShapeDtypeStruct((nbatch, 1, M), jnp.float32),
            jax.ShapeDtypeStruct((nbatch, 1, M), jnp.int32),
            jax.ShapeDtypeStruct((nbatch, 1, N), jnp.float32),
            jax.ShapeDtypeStruct((nbatch, 1, N), jnp.int32),
        ],
        compiler_params=pltpu.CompilerParams(
            dimension_semantics=("parallel", "arbitrary"),
        ),
        interpret=interpret,
    )(D)


_UNROLL = 8


def _stage2_sc(row_idx, row_e, col_idx, col_e, nbins):
    B, _, M = row_idx.shape
    P = 2 * B
    scale = 1.0 / (2.0 * B * M)
    mesh = plsc.VectorSubcoreMesh(core_axis_name="c", subcore_axis_name="s")

    @functools.partial(
        pl.kernel,
        mesh=mesh,
        out_type=jax.ShapeDtypeStruct((_LANES,), jnp.float32),
        scratch_types=[
            pltpu.VMEM((M,), jnp.int32),
            pltpu.VMEM((M,), jnp.float32),
            pltpu.VMEM((nbins,), jnp.float32),
            pltpu.VMEM((nbins,), jnp.float32),
            pltpu.VMEM((_LANES,), jnp.float32),
            pltpu.VMEM((P, _LANES), jnp.float32),
            pltpu.VMEM_SHARED((P, _LANES), jnp.float32),
        ],
        compiler_params=pltpu.CompilerParams(needs_layout_passes=False),
    )
    def k(ri_hbm, re_hbm, ci_hbm, ce_hbm, out_hbm, idx_v, val_v, s_v, c_v,
          acc_v, par_v, shared_v):
        cid = lax.axis_index("c")
        w = lax.axis_index("s")

        @pl.when((cid == 0) & (w < B))
        def _():
            pltpu.sync_copy(ri_hbm.at[w, 0], idx_v)
            pltpu.sync_copy(re_hbm.at[w, 0], val_v)

        @pl.when((cid == 0) & (w >= B) & (w < P))
        def _():
            pltpu.sync_copy(ci_hbm.at[w - B, 0], idx_v)
            pltpu.sync_copy(ce_hbm.at[w - B, 0], val_v)

        @pl.when((cid == 0) & (w < P))
        def _():

            zeros = jnp.zeros((_LANES,), jnp.float32)
            ones = jnp.ones((_LANES,), jnp.float32)

            @plsc.parallel_loop(0, nbins, _LANES, unroll=_UNROLL)
            def _zero(o):
                s_v[pl.ds(o, _LANES)] = zeros
                c_v[pl.ds(o, _LANES)] = zeros

            @plsc.parallel_loop(0, M, _LANES, unroll=_UNROLL)
            def _scat(o):
                iv = idx_v[pl.ds(o, _LANES)]
                vv = val_v[pl.ds(o, _LANES)]
                plsc.addupdate_scatter(s_v, [iv], vv)
                plsc.addupdate_scatter(c_v, [iv], ones)

            @plsc.parallel_loop(0, nbins, _LANES, unroll=_UNROLL, carry=zeros)
            def acc(o, a):
                sv = s_v[pl.ds(o, _LANES)]
                cv = c_v[pl.ds(o, _LANES)]
                return a + jnp.where(cv > 0.0, sv / jnp.maximum(cv, 1.0), 0.0)

            acc_v[...] = acc
            pltpu.sync_copy(acc_v, shared_v.at[w])

        plsc.subcore_barrier()

        @pl.when((cid == 0) & (w == 0))
        def _():
            pltpu.sync_copy(shared_v, par_v)
            total = par_v[0]
            for t in range(1, P):
                total = total + par_v[t]
            loss = 1.0 - jnp.sum(total) * scale
            acc_v[...] = jnp.zeros((_LANES,), jnp.float32) + loss
            pltpu.sync_copy(acc_v, out_hbm)

    return k(row_idx, row_e, col_idx, col_e)


def kernel(D):
    B, M, N = D.shape
    row_e, row_idx, col_e, col_idx = _stage1(D)
    out = _stage2_sc(row_idx, row_e, col_idx, col_e, max(M, N))
    return out[0]
